# Initial kernel scaffold; baseline (speedup 1.0000x reference)
import jax
import jax.numpy as jnp
from jax.experimental import pallas as pl


def kernel(inputs, weight):
    raise NotImplementedError("write your pallas kernel here")


# speedup vs baseline: 1.0645x; 1.0645x over previous
import jax
import jax.numpy as jnp
from jax.experimental import pallas as pl
from jax.experimental.pallas import tpu as pltpu
from jax.experimental.pallas import tpu_sc as plsc

NUM_CODES = 8192
DIM = 256
TOK_BLOCK = 1024
CODE_BLOCK = 512
COMMIT = 0.25
EPS = 1e-12

_GATHER_WINDOW = 128


def _wnorm_kernel(w_ref, wn_ref):
    w = w_ref[...]
    n = jnp.sqrt(jnp.sum(w * w, axis=1, keepdims=True))
    wn_ref[...] = w / jnp.maximum(n, EPS)


def _normalize_weight(weight):
    return pl.pallas_call(
        _wnorm_kernel,
        grid=(NUM_CODES // CODE_BLOCK,),
        in_specs=[pl.BlockSpec((CODE_BLOCK, DIM), lambda c: (c, 0))],
        out_specs=pl.BlockSpec((CODE_BLOCK, DIM), lambda c: (c, 0)),
        out_shape=jax.ShapeDtypeStruct((NUM_CODES, DIM), jnp.float32),
    )(weight)


def _vq_kernel(x_ref, wn_ref, idx_ref, minv_ref, runv, runi):
    c = pl.program_id(1)
    nc = pl.num_programs(1)
    x = x_ref[0]
    n = jnp.sqrt(jnp.sum(x * x, axis=0, keepdims=True))
    xh = x / jnp.maximum(n, EPS)
    xn2 = jnp.sum(xh * xh, axis=0, keepdims=True)
    wn = wn_ref[...]
    wn2 = jnp.sum(wn * wn, axis=1, keepdims=True)
    s = jax.lax.dot_general(
        wn.astype(jnp.bfloat16), xh.astype(jnp.bfloat16),
        (((1,), (0,)), ((), ())),
        preferred_element_type=jnp.float32)
    d = (xn2 + wn2) - 2.0 * s
    bv = jnp.min(d, axis=0, keepdims=True)
    ii = jax.lax.broadcasted_iota(jnp.int32, d.shape, 0)
    bi = jnp.min(jnp.where(d == bv, ii, jnp.int32(2**30)),
                 axis=0, keepdims=True) + c * CODE_BLOCK
    bvr = bv.astype(jnp.bfloat16).astype(jnp.float32)

    @pl.when(c == 0)
    def _():
        runv[...] = bvr
        runi[...] = bi

    @pl.when(c != 0)
    def _():
        upd = bv < runv[...]
        runv[...] = jnp.where(upd, bvr, runv[...])
        runi[...] = jnp.where(upd, bi, runi[...])

    @pl.when(c == nc - 1)
    def _():
        idx_ref[0] = runi[...]
        minv_ref[0] = runv[...]


def _vq_argmin(x3, wn):
    t_blocks = x3.shape[0]
    return pl.pallas_call(
        _vq_kernel,
        grid=(t_blocks, NUM_CODES // CODE_BLOCK),
        in_specs=[
            pl.BlockSpec((1, DIM, TOK_BLOCK), lambda t, c: (t, 0, 0)),
            pl.BlockSpec((CODE_BLOCK, DIM), lambda t, c: (c, 0)),
        ],
        out_specs=[
            pl.BlockSpec((1, 1, TOK_BLOCK), lambda t, c: (t, 0, 0)),
            pl.BlockSpec((1, 1, TOK_BLOCK), lambda t, c: (t, 0, 0)),
        ],
        out_shape=[
            jax.ShapeDtypeStruct((t_blocks, 1, TOK_BLOCK), jnp.int32),
            jax.ShapeDtypeStruct((t_blocks, 1, TOK_BLOCK), jnp.float32),
        ],
        scratch_shapes=[
            pltpu.VMEM((1, TOK_BLOCK), jnp.float32),
            pltpu.VMEM((1, TOK_BLOCK), jnp.int32),
        ],
        compiler_params=pltpu.CompilerParams(
            dimension_semantics=("parallel", "arbitrary"),
        ),
    )(x3, wn)


def _sc_gather(wn, idx_flat, num_tokens):
    idx2 = idx_flat.reshape(1, num_tokens)
    mesh = plsc.VectorSubcoreMesh(core_axis_name="core",
                                  subcore_axis_name="subcore")

    @pl.kernel(out_type=jax.ShapeDtypeStruct((num_tokens, DIM), wn.dtype),
               mesh=mesh)
    def _k(wn_hbm, i_hbm, o_hbm):
        def body(i_vmem, o_vmem):
            pltpu.sync_copy(wn_hbm.at[i_vmem.at[0]], o_vmem)

        pltpu.emit_pipeline(
            body,
            grid=(num_tokens // _GATHER_WINDOW,),
            in_specs=[pl.BlockSpec((1, _GATHER_WINDOW),
                                   index_map=lambda i: (0, i))],
            out_specs=[pl.BlockSpec((_GATHER_WINDOW, DIM),
                                    index_map=lambda i: (i, 0))],
            core_axis_name=("core", "subcore"),
            dimension_semantics=(pltpu.PARALLEL,),
        )(i_hbm, o_hbm)

    return _k(wn, idx2)


def kernel(inputs, weight):
    B, D, H, W = inputs.shape
    num_tokens = B * H * W
    x3 = inputs.reshape(B, D, H * W)

    wn = _normalize_weight(weight)
    idx3, minv3 = _vq_argmin(x3, wn)

    idx_flat = idx3.reshape(num_tokens)
    q = _sc_gather(wn, idx_flat, num_tokens)

    loss = (1.0 + COMMIT) * jnp.sum(minv3) / (num_tokens * D)
    quantized = jnp.transpose(q.reshape(B, H, W, D), (0, 3, 1, 2))
    encoding_indices = idx_flat[:, None]
    return (loss, quantized, encoding_indices)
